# baseline (speedup 1.0000x reference)
import functools

import jax
import jax.numpy as jnp
from jax import lax
from jax.experimental import pallas as pl
from jax.experimental.pallas import tpu as pltpu
from jax.experimental.pallas import tpu_sc as plsc

N = 1_600_000
S = 100_000
NTILES = 32
LANES = 16
NWORK = NTILES * LANES
SEG = 196
SP = NWORK * SEG
SEG_T = LANES * SEG
K = 512
R = 128
NB = 544
ACC_W = SEG * 4 * LANES


def _sc_compact_body(x2d_hbm, bounds_hbm, x_hbm,
                     bounds_v, xpad_v, xcomp_v, sem_a, sem_o):
    cid = lax.axis_index("c")
    sid = lax.axis_index("s")
    t = sid * 2 + cid
    t16 = t * LANES

    pltpu.sync_copy(bounds_hbm, bounds_v)
    iota = lax.iota(jnp.int32, LANES)
    lane_lo = plsc.load_gather(bounds_v, [t16 + iota])
    lane_hi = plsc.load_gather(bounds_v, [t16 + 1 + iota])
    astart = lane_lo & jnp.int32(-8)

    alo = astart[0]
    ahi = lane_hi[LANES - 1]
    n_r = (ahi - alo + (R - 1)) // R

    def _row_start(j):
        return pl.multiple_of(jnp.minimum(alo + j * R, N - R), 8)

    def _fire(j):
        slot = lax.rem(j, 2)
        rs = _row_start(j)
        pltpu.async_copy(x2d_hbm.at[pl.ds(rs, R)], xpad_v.at[slot], sem_a)

    @pl.when(n_r > 0)
    def _():
        _fire(0)

    def _compact(j, c):
        slot = lax.rem(j, 2)
        oslot = lax.rem(j, 4)

        @pl.when(j + 1 < n_r)
        def _():
            _fire(j + 1)

        pltpu.make_async_copy(x2d_hbm.at[pl.ds(0, R)],
                              xpad_v.at[slot], sem_a).wait()

        @pl.when(j >= 4)
        def _():
            pltpu.make_async_copy(xcomp_v.at[pl.ds(0, R * 3)],
                                  x_hbm.at[pl.ds(0, R * 3)], sem_o).wait()

        slotv = jnp.full((LANES,), slot, jnp.int32)
        for j2 in range(R // LANES):
            r = iota + j2 * LANES
            for c3 in range(3):
                cvec = jnp.full((LANES,), c3, jnp.int32)
                vv = plsc.load_gather(xpad_v, [slotv, r, cvec])
                plsc.store_scatter(xcomp_v, [oslot * (R * 3) + r * 3 + c3], vv)
        rs = _row_start(j)
        pltpu.async_copy(xcomp_v.at[pl.ds(oslot * (R * 3), R * 3)],
                         x_hbm.at[pl.ds(pl.multiple_of(rs * 3, 8), R * 3)],
                         sem_o)
        return c
    lax.fori_loop(0, n_r, _compact, 0)

    def _drain(j, c):
        pltpu.make_async_copy(xcomp_v.at[pl.ds(0, R * 3)],
                              x_hbm.at[pl.ds(0, R * 3)], sem_o).wait()
        return c
    lax.fori_loop(0, jnp.minimum(n_r, 4), _drain, 0)


def _sc_body(x_hbm, ids_hbm, bounds_hbm, params_hbm, out_hbm,
             bounds_v, params_v, ids_v, x_v, acc, outv, sem, sem2):
    cid = lax.axis_index("c")
    sid = lax.axis_index("s")
    t = sid * 2 + cid
    t16 = t * LANES

    pltpu.sync_copy(bounds_hbm, bounds_v)
    pltpu.sync_copy(params_hbm, params_v)

    iota = lax.iota(jnp.int32, LANES)
    lane_lo = plsc.load_gather(bounds_v, [t16 + iota])
    lane_hi = plsc.load_gather(bounds_v, [t16 + 1 + iota])
    astart = lane_lo & jnp.int32(-8)
    seg_base = (t16 + iota) * SEG
    maxlen = jnp.max(lane_hi - astart)
    nchunks = (maxlen + (K - 1)) // K

    def _zero(i, c):
        acc[pl.ds(i * LANES, LANES)] = jnp.zeros((LANES,), jnp.float32)
        return c
    lax.fori_loop(0, ACC_W // LANES, _zero, 0)

    iota_k = iota * K
    iota_k3 = iota * (K * 3)

    def _chunk(ci, c):
        s0 = ci * K
        dvec = jnp.minimum(astart + s0, N - K)
        copies = []
        for l in range(LANES):
            dst = pl.multiple_of(dvec[l], 8)
            h1 = pltpu.async_copy(ids_hbm.at[pl.ds(dst, K)],
                                  ids_v.at[pl.ds(l * K, K)], sem)
            h2 = pltpu.async_copy(x_hbm.at[pl.ds(pl.multiple_of(dst * 3, 8), K * 3)],
                                  x_v.at[pl.ds(l * (K * 3), K * 3)], sem2)
            copies.append((h1, h2))
        for h1, h2 in copies:
            h1.wait()
            h2.wait()

        mlo = jnp.maximum(lane_lo, astart + s0)

        def _step(s, cc):
            ids = plsc.load_gather(ids_v, [iota_k + s])
            gx = dvec + s
            valid = (gx >= mlo) & (gx < lane_hi)
            lid = jnp.clip(ids - seg_base, 0, SEG - 1)
            aidx = lid * (4 * LANES) + iota
            xb = iota_k3 + s * 3
            xv = plsc.load_gather(x_v, [xb])
            yv = plsc.load_gather(x_v, [xb + 1])
            zv = plsc.load_gather(x_v, [xb + 2])
            zero = jnp.zeros((LANES,), jnp.float32)
            plsc.addupdate_scatter(acc, [aidx], jnp.where(valid, xv, zero))
            plsc.addupdate_scatter(acc, [aidx + LANES], jnp.where(valid, yv, zero))
            plsc.addupdate_scatter(acc, [aidx + 2 * LANES], jnp.where(valid, zv, zero))
            plsc.addupdate_scatter(acc, [aidx + 3 * LANES],
                                   jnp.where(valid, jnp.ones((LANES,), jnp.float32), zero))
            return cc
        lax.fori_loop(0, K, _step, 0, unroll=4)
        return c
    lax.fori_loop(0, nchunks, _chunk, 0)

    pv = params_v[pl.ds(0, LANES)]
    g0 = pv[0]
    g1 = pv[1]
    g2 = pv[2]
    b0 = pv[3]
    b1 = pv[4]
    b2 = pv[5]
    w0 = pv[6]
    w1 = pv[7]
    w2 = pv[8]
    bb = pv[9]

    def _head(lid, c):
        base = lid * (4 * LANES)
        sx = acc[pl.ds(base, LANES)]
        sy = acc[pl.ds(base + LANES, LANES)]
        sz = acc[pl.ds(base + 2 * LANES, LANES)]
        cn = acc[pl.ds(base + 3 * LANES, LANES)]
        cn1 = jnp.maximum(cn, 1.0)
        inv_n = 1.0 / cn1
        inv_n = inv_n * (2.0 - cn1 * inv_n)
        mx = sx * inv_n
        my = sy * inv_n
        mz = sz * inv_n
        mu = (mx + my + mz) * jnp.float32(1.0 / 3.0)
        dx = mx - mu
        dy = my - mu
        dz = mz - mu
        v = (dx * dx + dy * dy + dz * dz) * jnp.float32(1.0 / 3.0) + 1e-5
        vi = plsc.bitcast(v, jnp.int32)
        y = plsc.bitcast(jnp.int32(0x5F3759DF) - (vi >> 1), jnp.float32)
        for _ in range(4):
            y = y * (1.5 - 0.5 * v * y * y)
        h0 = dx * y * g0 + b0
        h1 = dy * y * g1 + b1
        h2 = dz * y * g2 + b2
        pred = h0 * w0 + h1 * w1 + h2 * w2 + bb
        plsc.store_scatter(outv, [iota * SEG + lid], pred)
        return c
    lax.fori_loop(0, SEG, _head, 0)

    pltpu.sync_copy(outv, out_hbm.at[pl.ds(t * SEG_T, SEG_T)])


@jax.jit
def _run(x2d, ids, bounds, params):
    mesh = plsc.VectorSubcoreMesh(core_axis_name="c", subcore_axis_name="s")
    compact_fn = pl.kernel(
        _sc_compact_body,
        out_type=jax.ShapeDtypeStruct((N * 3,), jnp.float32),
        mesh=mesh,
        compiler_params=pltpu.CompilerParams(needs_layout_passes=False),
        scratch_types=[
            pltpu.VMEM((NB,), jnp.int32),
            pltpu.VMEM((2, R, 3), jnp.float32),
            pltpu.VMEM((4 * R * 3,), jnp.float32),
            pltpu.SemaphoreType.DMA,
            pltpu.SemaphoreType.DMA,
        ],
    )
    x_flat = compact_fn(x2d, bounds)
    fn = pl.kernel(
        _sc_body,
        out_type=jax.ShapeDtypeStruct((SP,), jnp.float32),
        mesh=mesh,
        compiler_params=pltpu.CompilerParams(needs_layout_passes=False),
        scratch_types=[
            pltpu.VMEM((NB,), jnp.int32),
            pltpu.VMEM((LANES,), jnp.float32),
            pltpu.VMEM((LANES * K,), jnp.int32),
            pltpu.VMEM((LANES * K * 3,), jnp.float32),
            pltpu.VMEM((ACC_W,), jnp.float32),
            pltpu.VMEM((SEG_T,), jnp.float32),
            pltpu.SemaphoreType.DMA,
            pltpu.SemaphoreType.DMA,
        ],
    )
    return fn(x_flat, ids, bounds, params)


def kernel(x_t, batch, ln_scale, ln_bias, W, b):
    ids = batch.astype(jnp.int32)
    edges = jnp.arange(NWORK + 1, dtype=jnp.int32) * SEG
    bounds = jnp.searchsorted(ids, edges, side="left").astype(jnp.int32)
    bounds = jnp.pad(bounds, (0, NB - (NWORK + 1)))
    params = jnp.concatenate([
        ln_scale.astype(jnp.float32),
        ln_bias.astype(jnp.float32),
        W.reshape(-1).astype(jnp.float32),
        b.reshape(-1).astype(jnp.float32),
        jnp.zeros((6,), jnp.float32),
    ])
    out = _run(x_t.astype(jnp.float32), ids, bounds, params)
    return out[:S].reshape(S, 1)


# speedup vs baseline: 1.0081x; 1.0081x over previous
import functools

import jax
import jax.numpy as jnp
from jax import lax
from jax.experimental import pallas as pl
from jax.experimental.pallas import tpu as pltpu
from jax.experimental.pallas import tpu_sc as plsc

N = 1_600_000
S = 100_000
NTILES = 32
LANES = 16
NWORK = NTILES * LANES
SEG = 196
SP = NWORK * SEG
SEG_T = LANES * SEG
K = 512
R = 128
NB = 544
ACC_W = SEG * 4 * LANES


def _sc_compact_body(x2d_hbm, bounds_hbm, x_hbm,
                     bounds_v, xpad_v, xcomp_v, sem_a, sem_o):
    cid = lax.axis_index("c")
    sid = lax.axis_index("s")
    t = sid * 2 + cid
    t16 = t * LANES

    pltpu.sync_copy(bounds_hbm, bounds_v)
    iota = lax.iota(jnp.int32, LANES)
    lane_lo = plsc.load_gather(bounds_v, [t16 + iota])
    lane_hi = plsc.load_gather(bounds_v, [t16 + 1 + iota])
    astart = lane_lo & jnp.int32(-8)

    alo = astart[0]
    ahi = lane_hi[LANES - 1]
    n_r = (ahi - alo + (R - 1)) // R

    def _row_start(j):
        return pl.multiple_of(jnp.minimum(alo + j * R, N - R), 8)

    def _fire(j):
        slot = lax.rem(j, 2)
        rs = _row_start(j)
        pltpu.async_copy(x2d_hbm.at[pl.ds(rs, R)], xpad_v.at[slot], sem_a)

    @pl.when(n_r > 0)
    def _():
        _fire(0)

    def _compact(j, c):
        slot = lax.rem(j, 2)
        oslot = lax.rem(j, 4)

        @pl.when(j + 1 < n_r)
        def _():
            _fire(j + 1)

        pltpu.make_async_copy(x2d_hbm.at[pl.ds(0, R)],
                              xpad_v.at[slot], sem_a).wait()

        @pl.when(j >= 4)
        def _():
            pltpu.make_async_copy(xcomp_v.at[pl.ds(0, R * 3)],
                                  x_hbm.at[pl.ds(0, R * 3)], sem_o).wait()

        slotv = jnp.full((LANES,), slot, jnp.int32)
        for j2 in range(R // LANES):
            r = iota + j2 * LANES
            for c3 in range(3):
                cvec = jnp.full((LANES,), c3, jnp.int32)
                vv = plsc.load_gather(xpad_v, [slotv, r, cvec])
                plsc.store_scatter(xcomp_v, [oslot * (R * 3) + r * 3 + c3], vv)
        rs = _row_start(j)
        pltpu.async_copy(xcomp_v.at[pl.ds(oslot * (R * 3), R * 3)],
                         x_hbm.at[pl.ds(pl.multiple_of(rs * 3, 8), R * 3)],
                         sem_o)
        return c
    lax.fori_loop(0, n_r, _compact, 0)

    def _drain(j, c):
        pltpu.make_async_copy(xcomp_v.at[pl.ds(0, R * 3)],
                              x_hbm.at[pl.ds(0, R * 3)], sem_o).wait()
        return c
    lax.fori_loop(0, jnp.minimum(n_r, 4), _drain, 0)


def _sc_body(x_hbm, ids_hbm, bounds_hbm, params_hbm, out_hbm,
             bounds_v, params_v, ids_v, x_v, acc, outv, sem, sem2):
    cid = lax.axis_index("c")
    sid = lax.axis_index("s")
    t = sid * 2 + cid
    t16 = t * LANES

    pltpu.sync_copy(bounds_hbm, bounds_v)
    pltpu.sync_copy(params_hbm, params_v)

    iota = lax.iota(jnp.int32, LANES)
    lane_lo = plsc.load_gather(bounds_v, [t16 + iota])
    lane_hi = plsc.load_gather(bounds_v, [t16 + 1 + iota])
    astart = lane_lo & jnp.int32(-8)
    seg_base = (t16 + iota) * SEG
    maxlen = jnp.max(lane_hi - astart)
    nchunks = (maxlen + (K - 1)) // K

    def _zero(i, c):
        acc[pl.ds(i * LANES, LANES)] = jnp.zeros((LANES,), jnp.float32)
        return c
    lax.fori_loop(0, ACC_W // LANES, _zero, 0)

    iota_k = iota * K
    iota_k3 = iota * (K * 3)

    def _fire_b(ci):
        slot = lax.rem(ci, 2)
        dvec = jnp.minimum(astart + ci * K, N - K)
        for l in range(LANES):
            dst = pl.multiple_of(dvec[l], 8)
            pltpu.async_copy(
                ids_hbm.at[pl.ds(dst, K)],
                ids_v.at[pl.ds(slot * (LANES * K) + l * K, K)], sem)
            pltpu.async_copy(
                x_hbm.at[pl.ds(pl.multiple_of(dst * 3, 8), K * 3)],
                x_v.at[pl.ds(slot * (LANES * K * 3) + l * (K * 3), K * 3)],
                sem2)

    @pl.when(nchunks > 0)
    def _():
        _fire_b(0)

    def _chunk(ci, c):
        s0 = ci * K
        slot = lax.rem(ci, 2)
        base_i = slot * (LANES * K)
        base_x = slot * (LANES * K * 3)

        @pl.when(ci + 1 < nchunks)
        def _():
            _fire_b(ci + 1)

        for l in range(LANES):
            pltpu.make_async_copy(
                ids_hbm.at[pl.ds(0, K)],
                ids_v.at[pl.ds(base_i + l * K, K)], sem).wait()
            pltpu.make_async_copy(
                x_hbm.at[pl.ds(0, K * 3)],
                x_v.at[pl.ds(base_x + l * (K * 3), K * 3)], sem2).wait()

        dvec = jnp.minimum(astart + s0, N - K)
        mlo = jnp.maximum(lane_lo, astart + s0)

        def _step(s, cc):
            ids = plsc.load_gather(ids_v, [base_i + iota_k + s])
            gx = dvec + s
            valid = (gx >= mlo) & (gx < lane_hi)
            lid = jnp.clip(ids - seg_base, 0, SEG - 1)
            aidx = lid * (4 * LANES) + iota
            xb = base_x + iota_k3 + s * 3
            xv = plsc.load_gather(x_v, [xb])
            yv = plsc.load_gather(x_v, [xb + 1])
            zv = plsc.load_gather(x_v, [xb + 2])
            zero = jnp.zeros((LANES,), jnp.float32)
            plsc.addupdate_scatter(acc, [aidx], jnp.where(valid, xv, zero))
            plsc.addupdate_scatter(acc, [aidx + LANES], jnp.where(valid, yv, zero))
            plsc.addupdate_scatter(acc, [aidx + 2 * LANES], jnp.where(valid, zv, zero))
            plsc.addupdate_scatter(acc, [aidx + 3 * LANES],
                                   jnp.where(valid, jnp.ones((LANES,), jnp.float32), zero))
            return cc
        lax.fori_loop(0, K, _step, 0, unroll=4)
        return c
    lax.fori_loop(0, nchunks, _chunk, 0)

    pv = params_v[pl.ds(0, LANES)]
    g0 = pv[0]
    g1 = pv[1]
    g2 = pv[2]
    b0 = pv[3]
    b1 = pv[4]
    b2 = pv[5]
    w0 = pv[6]
    w1 = pv[7]
    w2 = pv[8]
    bb = pv[9]

    def _head(lid, c):
        base = lid * (4 * LANES)
        sx = acc[pl.ds(base, LANES)]
        sy = acc[pl.ds(base + LANES, LANES)]
        sz = acc[pl.ds(base + 2 * LANES, LANES)]
        cn = acc[pl.ds(base + 3 * LANES, LANES)]
        cn1 = jnp.maximum(cn, 1.0)
        inv_n = 1.0 / cn1
        inv_n = inv_n * (2.0 - cn1 * inv_n)
        mx = sx * inv_n
        my = sy * inv_n
        mz = sz * inv_n
        mu = (mx + my + mz) * jnp.float32(1.0 / 3.0)
        dx = mx - mu
        dy = my - mu
        dz = mz - mu
        v = (dx * dx + dy * dy + dz * dz) * jnp.float32(1.0 / 3.0) + 1e-5
        vi = plsc.bitcast(v, jnp.int32)
        y = plsc.bitcast(jnp.int32(0x5F3759DF) - (vi >> 1), jnp.float32)
        for _ in range(4):
            y = y * (1.5 - 0.5 * v * y * y)
        h0 = dx * y * g0 + b0
        h1 = dy * y * g1 + b1
        h2 = dz * y * g2 + b2
        pred = h0 * w0 + h1 * w1 + h2 * w2 + bb
        plsc.store_scatter(outv, [iota * SEG + lid], pred)
        return c
    lax.fori_loop(0, SEG, _head, 0)

    pltpu.sync_copy(outv, out_hbm.at[pl.ds(t * SEG_T, SEG_T)])


@jax.jit
def _run(x2d, ids, bounds, params):
    mesh = plsc.VectorSubcoreMesh(core_axis_name="c", subcore_axis_name="s")
    compact_fn = pl.kernel(
        _sc_compact_body,
        out_type=jax.ShapeDtypeStruct((N * 3,), jnp.float32),
        mesh=mesh,
        compiler_params=pltpu.CompilerParams(needs_layout_passes=False),
        scratch_types=[
            pltpu.VMEM((NB,), jnp.int32),
            pltpu.VMEM((2, R, 3), jnp.float32),
            pltpu.VMEM((4 * R * 3,), jnp.float32),
            pltpu.SemaphoreType.DMA,
            pltpu.SemaphoreType.DMA,
        ],
    )
    x_flat = compact_fn(x2d, bounds)
    fn = pl.kernel(
        _sc_body,
        out_type=jax.ShapeDtypeStruct((SP,), jnp.float32),
        mesh=mesh,
        compiler_params=pltpu.CompilerParams(needs_layout_passes=False),
        scratch_types=[
            pltpu.VMEM((NB,), jnp.int32),
            pltpu.VMEM((LANES,), jnp.float32),
            pltpu.VMEM((2 * LANES * K,), jnp.int32),
            pltpu.VMEM((2 * LANES * K * 3,), jnp.float32),
            pltpu.VMEM((ACC_W,), jnp.float32),
            pltpu.VMEM((SEG_T,), jnp.float32),
            pltpu.SemaphoreType.DMA,
            pltpu.SemaphoreType.DMA,
        ],
    )
    return fn(x_flat, ids, bounds, params)


def kernel(x_t, batch, ln_scale, ln_bias, W, b):
    ids = batch.astype(jnp.int32)
    edges = jnp.arange(NWORK + 1, dtype=jnp.int32) * SEG
    bounds = jnp.searchsorted(ids, edges, side="left").astype(jnp.int32)
    bounds = jnp.pad(bounds, (0, NB - (NWORK + 1)))
    params = jnp.concatenate([
        ln_scale.astype(jnp.float32),
        ln_bias.astype(jnp.float32),
        W.reshape(-1).astype(jnp.float32),
        b.reshape(-1).astype(jnp.float32),
        jnp.zeros((6,), jnp.float32),
    ])
    out = _run(x_t.astype(jnp.float32), ids, bounds, params)
    return out[:S].reshape(S, 1)
